# baseline (speedup 1.0000x reference)
import functools

import jax
import jax.numpy as jnp
from jax import lax
from jax.experimental import pallas as pl
from jax.experimental.pallas import tpu as pltpu
from jax.experimental.pallas import tpu_sc as plsc

N_NODES = 10000
N_HEDGES = 5000
NNZ = 320000
D = 128
W = 64

NC = 2
NS = 16
NW = NC * NS
L = 16

CH = 128
EPT = NNZ // NS
NCHUNK = 158
TOTCH = 160
NCH_H = 80
N_PAD = 10240
E_PAD = 5120
PAD_NODE = N_PAD - 1
PAD_HEDGE = E_PAD - 1
RPE = E_PAD // NS
RPN = N_PAD // NS

_mesh = functools.partial(
    plsc.VectorSubcoreMesh, core_axis_name="c", subcore_axis_name="s",
    num_cores=NC, num_subcores=NS)
_sc_params = pltpu.CompilerParams(needs_layout_passes=False,
                                  use_tc_tiling_on_sc=False)


def _hist_body(node_hbm, hedge_hbm, zeros_hbm, dvp_hbm, dep_hbm,
               nidx_v, hidx_v, histn_v, histe_v):
    c = lax.axis_index("c")
    s = lax.axis_index("s")
    w = s * NC + c
    pltpu.sync_copy(zeros_hbm, histn_v)
    pltpu.sync_copy(zeros_hbm.at[pl.ds(0, E_PAD // L)], histe_v)
    pltpu.sync_copy(node_hbm.at[s, pl.ds(c * NCH_H, NCH_H)], nidx_v)
    pltpu.sync_copy(hedge_hbm.at[s, pl.ds(c * NCH_H, NCH_H)], hidx_v)
    ones = jnp.full((L,), 1.0, jnp.float32)

    def body(i, carry):
        j = i // (CH // L)
        col = (i % (CH // L)) * L
        nv = nidx_v[j, pl.ds(col, L)]
        plsc.addupdate_scatter(histn_v, [nv >> 4, nv & 15], ones)
        hv = hidx_v[j, pl.ds(col, L)]
        plsc.addupdate_scatter(histe_v, [hv >> 4, hv & 15], ones)
        return carry

    lax.fori_loop(0, NCH_H * (CH // L), body, 0)
    pltpu.sync_copy(histn_v, dvp_hbm.at[w])
    pltpu.sync_copy(histe_v, dep_hbm.at[w])


def _hist_call(node_p, hedge_p, zeros1):
    return pl.kernel(
        _hist_body,
        out_type=(jax.ShapeDtypeStruct((NW, N_PAD // L, L), jnp.float32),
                  jax.ShapeDtypeStruct((NW, E_PAD // L, L), jnp.float32)),
        mesh=_mesh(),
        compiler_params=_sc_params,
        scratch_types=[
            pltpu.VMEM((NCH_H, CH), jnp.int32),
            pltpu.VMEM((NCH_H, CH), jnp.int32),
            pltpu.VMEM((N_PAD // L, L), jnp.float32),
            pltpu.VMEM((E_PAD // L, L), jnp.float32),
        ],
    )(node_p, hedge_p, zeros1)


ZB = 64


def _main_body(y_hbm, xt_hbm, dem_hbm, dvis_hbm, nidx_hbm, hidx_hbm,
               zeros_hbm, out_hbm,
               nidx_v, hidx_v, bufa, bufb, bufc, xe_sh, ya_sh, sema, semb):
    c = lax.axis_index("c")
    s = lax.axis_index("s")
    ba = bufa.at[pl.ds(0, ZB)]
    bb = bufb.at[pl.ds(0, ZB)]

    pltpu.sync_copy(nidx_hbm.at[s], nidx_v)
    pltpu.sync_copy(hidx_hbm.at[s], hidx_v)
    for k in range(RPN // ZB):
        r0 = s * RPN + k * ZB
        pltpu.sync_copy(y_hbm.at[c, pl.ds(r0, ZB)], ya_sh.at[pl.ds(r0, ZB)])
    pltpu.sync_copy(zeros_hbm, ba)
    for k in range(RPE // ZB):
        pltpu.sync_copy(ba, xe_sh.at[pl.ds(s * RPE + k * ZB, ZB)])
    plsc.subcore_barrier()

    pltpu.async_copy(ya_sh.at[nidx_v.at[0]], bufa, sema)

    def step1(m, carry):
        m2 = 2 * m
        pltpu.async_copy(ya_sh.at[nidx_v.at[m2 + 1]], bufb, semb)
        pltpu.make_async_copy(ya_sh.at[nidx_v.at[m2]], bufa, sema).wait()
        pltpu.sync_copy(bufa, xe_sh.at[hidx_v.at[m2]], add=True)
        pltpu.async_copy(ya_sh.at[nidx_v.at[m2 + 2]], bufa, sema)
        pltpu.make_async_copy(ya_sh.at[nidx_v.at[m2 + 1]], bufb, semb).wait()
        pltpu.sync_copy(bufb, xe_sh.at[hidx_v.at[m2 + 1]], add=True)
        return carry

    lax.fori_loop(0, NCHUNK // 2, step1, 0)
    pltpu.make_async_copy(ya_sh.at[nidx_v.at[NCHUNK]], bufa, sema).wait()
    plsc.subcore_barrier()

    def scale(i, carry):
        r = i // (W // L)
        col = (i % (W // L)) * L
        bufa[r, pl.ds(col, L)] = bufa[r, pl.ds(col, L)] * bufb[r, pl.ds(col, L)]
        return carry

    for k in range(RPE // ZB):
        e0 = s * RPE + k * ZB
        pltpu.sync_copy(xe_sh.at[pl.ds(e0, ZB)], ba)
        pltpu.sync_copy(dem_hbm.at[pl.ds(e0, ZB)], bb)
        lax.fori_loop(0, ZB * (W // L), scale, 0)
        pltpu.sync_copy(ba, xe_sh.at[pl.ds(e0, ZB)])

    pltpu.sync_copy(zeros_hbm, ba)
    for k in range(RPN // ZB):
        pltpu.sync_copy(ba, ya_sh.at[pl.ds(s * RPN + k * ZB, ZB)])
    plsc.subcore_barrier()

    pltpu.async_copy(xe_sh.at[hidx_v.at[0]], bufa, sema)

    def step2(m, carry):
        m2 = 2 * m
        pltpu.async_copy(xe_sh.at[hidx_v.at[m2 + 1]], bufb, semb)
        pltpu.make_async_copy(xe_sh.at[hidx_v.at[m2]], bufa, sema).wait()
        pltpu.sync_copy(bufa, ya_sh.at[nidx_v.at[m2]], add=True)
        pltpu.async_copy(xe_sh.at[hidx_v.at[m2 + 2]], bufa, sema)
        pltpu.make_async_copy(xe_sh.at[hidx_v.at[m2 + 1]], bufb, semb).wait()
        pltpu.sync_copy(bufb, ya_sh.at[nidx_v.at[m2 + 1]], add=True)
        return carry

    lax.fori_loop(0, NCHUNK // 2, step2, 0)
    pltpu.make_async_copy(xe_sh.at[hidx_v.at[NCHUNK]], bufa, sema).wait()
    plsc.subcore_barrier()

    def finish(i, carry):
        r = i // (W // L)
        col = (i % (W // L)) * L
        bufa[r, pl.ds(col, L)] = jnp.maximum(
            bufa[r, pl.ds(col, L)] * bufc[r, pl.ds(col, L)]
            + bufb[r, pl.ds(col, L)], 0.0)
        return carry

    for k in range(RPN // ZB):
        n0 = s * RPN + k * ZB
        pltpu.sync_copy(ya_sh.at[pl.ds(n0, ZB)], ba)
        pltpu.sync_copy(xt_hbm.at[c, pl.ds(n0, ZB)], bb)
        pltpu.sync_copy(dvis_hbm.at[pl.ds(n0, ZB)], bufc)
        lax.fori_loop(0, ZB * (W // L), finish, 0)
        pltpu.sync_copy(ba, out_hbm.at[c, pl.ds(n0, ZB)])


def _main_call(ystack, xtstack, demat, dvis2d, nidx, hidx):
    zeros2 = jnp.zeros((ZB, W), jnp.float32)
    return pl.kernel(
        _main_body,
        out_type=jax.ShapeDtypeStruct((NC, N_PAD, W), jnp.float32),
        mesh=_mesh(),
        compiler_params=_sc_params,
        scratch_types=[
            pltpu.VMEM((TOTCH, CH), jnp.int32),
            pltpu.VMEM((TOTCH, CH), jnp.int32),
            pltpu.VMEM((CH, W), jnp.float32),
            pltpu.VMEM((CH, W), jnp.float32),
            pltpu.VMEM((ZB, W), jnp.float32),
            pltpu.VMEM_SHARED((E_PAD, W), jnp.float32),
            pltpu.VMEM_SHARED((N_PAD, W), jnp.float32),
            pltpu.SemaphoreType.DMA,
            pltpu.SemaphoreType.DMA,
        ],
    )(ystack, xtstack, demat, dvis2d, nidx, hidx, zeros2)


def _tc1_body(feat_ref, win_ref, bin_ref, w2_ref, b2_ref, dvp_ref, dep_ref,
              xt_ref, y_ref, dvis_ref, dem_ref):
    x = feat_ref[...]
    h = lax.dot_general(x, win_ref[...], (((1,), (1,)), ((), ())),
                        preferred_element_type=jnp.float32) + bin_ref[...]
    h = jnp.where(h >= 0, h, 0.2 * h)
    xt = lax.dot_general(h, w2_ref[...], (((1,), (1,)), ((), ())),
                         preferred_element_type=jnp.float32) + b2_ref[...]
    dv = jnp.sum(dvp_ref[...], axis=0)
    dvis = jnp.where(dv > 0, lax.rsqrt(dv), 0.0)
    xt_ref[0] = xt[:, :W]
    xt_ref[1] = xt[:, W:]
    y = xt * dvis[:, None]
    y_ref[0] = y[:, :W]
    y_ref[1] = y[:, W:]
    dvis_ref[...] = jnp.broadcast_to(dvis[:, None], dvis_ref.shape)
    de = jnp.sum(dep_ref[...], axis=0)
    deinv = jnp.where(de > 0, 1.0 / de, 0.0)
    dem_ref[...] = jnp.broadcast_to(deinv[:, None], (E_PAD // 10, W))


def _tc1_call(feat_p, w_in, b_in, w2, b2, dvp, dep):
    blk = 1024
    eblk = E_PAD // 10
    grid = N_PAD // blk
    return pl.pallas_call(
        _tc1_body,
        grid=(grid,),
        in_specs=[
            pl.BlockSpec((blk, D), lambda i: (i, 0)),
            pl.BlockSpec((D, D), lambda i: (0, 0)),
            pl.BlockSpec((1, D), lambda i: (0, 0)),
            pl.BlockSpec((D, D), lambda i: (0, 0)),
            pl.BlockSpec((1, D), lambda i: (0, 0)),
            pl.BlockSpec((NW, blk), lambda i: (0, i)),
            pl.BlockSpec((NW, eblk), lambda i: (0, i)),
        ],
        out_specs=[pl.BlockSpec((NC, blk, W), lambda i: (0, i, 0)),
                   pl.BlockSpec((NC, blk, W), lambda i: (0, i, 0)),
                   pl.BlockSpec((blk, W), lambda i: (i, 0)),
                   pl.BlockSpec((eblk, W), lambda i: (i, 0))],
        out_shape=[jax.ShapeDtypeStruct((NC, N_PAD, W), jnp.float32),
                   jax.ShapeDtypeStruct((NC, N_PAD, W), jnp.float32),
                   jax.ShapeDtypeStruct((N_PAD, W), jnp.float32),
                   jax.ShapeDtypeStruct((E_PAD, W), jnp.float32)],
    )(feat_p, w_in, b_in, w2, b2, dvp, dep)


def _pad_idx(idx, pad_val):
    cols = TOTCH * CH - EPT
    return jnp.concatenate(
        [idx.reshape(NS, EPT),
         jnp.full((NS, cols), pad_val, jnp.int32)],
        axis=1).reshape(NS, TOTCH, CH)


def kernel(feat, node_idx, hedge_idx, W_in, b_in, W1, b1, W2, b2):
    f32 = jnp.float32
    feat_p = jnp.zeros((N_PAD, D), f32).at[:N_NODES, :].set(feat)
    node_p = _pad_idx(node_idx, PAD_NODE)
    hedge_p = _pad_idx(hedge_idx, PAD_HEDGE)
    zeros1 = jnp.zeros((N_PAD // L, L), f32)

    dvp, dep = _hist_call(node_p, hedge_p, zeros1)
    dvp = dvp.reshape(NW, N_PAD)
    dep = dep.reshape(NW, E_PAD)
    xtstack, ystack, dvis2d, demat = _tc1_call(
        feat_p, W_in, b_in.reshape(1, D), W2, b2.reshape(1, D), dvp, dep)
    pn = _main_call(ystack, xtstack, demat, dvis2d, node_p, hedge_p)
    return jnp.concatenate([pn[0], pn[1]], axis=1)[:N_NODES]


# speedup vs baseline: 1.2443x; 1.2443x over previous
import functools

import jax
import jax.numpy as jnp
from jax import lax
from jax.experimental import pallas as pl
from jax.experimental.pallas import tpu as pltpu
from jax.experimental.pallas import tpu_sc as plsc

N_NODES = 10000
N_HEDGES = 5000
NNZ = 320000
D = 128
W = 64

NC = 2
NS = 16
NW = NC * NS
L = 16

CH = 128
EPT = NNZ // NS
NCHUNK = 159
TOTCH = 160
NCH_H = 80
N_PAD = 10240
E_PAD = 5120
PAD_NODE = N_PAD - 1
PAD_HEDGE = E_PAD - 1
RPE = E_PAD // NS
RPN = N_PAD // NS

_mesh = functools.partial(
    plsc.VectorSubcoreMesh, core_axis_name="c", subcore_axis_name="s",
    num_cores=NC, num_subcores=NS)
_sc_params = pltpu.CompilerParams(needs_layout_passes=False,
                                  use_tc_tiling_on_sc=False)


def _hist_body(node_hbm, hedge_hbm, zeros_hbm, dvp_hbm, dep_hbm,
               nidx_v, hidx_v, histn_v, histe_v):
    c = lax.axis_index("c")
    s = lax.axis_index("s")
    w = s * NC + c
    pltpu.sync_copy(zeros_hbm, histn_v)
    pltpu.sync_copy(zeros_hbm.at[pl.ds(0, E_PAD // L)], histe_v)
    pltpu.sync_copy(node_hbm.at[s, pl.ds(c * NCH_H, NCH_H)], nidx_v)
    pltpu.sync_copy(hedge_hbm.at[s, pl.ds(c * NCH_H, NCH_H)], hidx_v)
    ones = jnp.full((L,), 1.0, jnp.float32)

    def body(i, carry):
        j = i // (CH // L)
        col = (i % (CH // L)) * L
        nv = nidx_v[j, pl.ds(col, L)]
        plsc.addupdate_scatter(histn_v, [nv >> 4, nv & 15], ones)
        hv = hidx_v[j, pl.ds(col, L)]
        plsc.addupdate_scatter(histe_v, [hv >> 4, hv & 15], ones)
        return carry

    lax.fori_loop(0, NCH_H * (CH // L), body, 0)
    pltpu.sync_copy(histn_v, dvp_hbm.at[w])
    pltpu.sync_copy(histe_v, dep_hbm.at[w])


def _hist_call(node_p, hedge_p, zeros1):
    return pl.kernel(
        _hist_body,
        out_type=(jax.ShapeDtypeStruct((NW, N_PAD // L, L), jnp.float32),
                  jax.ShapeDtypeStruct((NW, E_PAD // L, L), jnp.float32)),
        mesh=_mesh(),
        compiler_params=_sc_params,
        scratch_types=[
            pltpu.VMEM((NCH_H, CH), jnp.int32),
            pltpu.VMEM((NCH_H, CH), jnp.int32),
            pltpu.VMEM((N_PAD // L, L), jnp.float32),
            pltpu.VMEM((E_PAD // L, L), jnp.float32),
        ],
    )(node_p, hedge_p, zeros1)


ZB = 64


def _main_body(y_hbm, dem_hbm, nidx_hbm, hidx_hbm, zeros_hbm, out_hbm,
               nidx_v, hidx_v, bufa, bufb, bufc, xe_sh, ya_sh,
               gsa, gsb, gsc, ssa, ssb, ssc):
    c = lax.axis_index("c")
    s = lax.axis_index("s")
    ba = bufa.at[pl.ds(0, ZB)]
    bb = bufb.at[pl.ds(0, ZB)]

    pltpu.sync_copy(nidx_hbm.at[s], nidx_v)
    pltpu.sync_copy(hidx_hbm.at[s], hidx_v)
    for k in range(RPN // ZB):
        r0 = s * RPN + k * ZB
        pltpu.sync_copy(y_hbm.at[c, pl.ds(r0, ZB)], ya_sh.at[pl.ds(r0, ZB)])
    pltpu.sync_copy(zeros_hbm, ba)
    for k in range(RPE // ZB):
        pltpu.sync_copy(ba, xe_sh.at[pl.ds(s * RPE + k * ZB, ZB)])
    plsc.subcore_barrier()

    def sweep(src_sh, dst_sh, gidx_v, sidx_v):
        bufs = (bufa, bufb, bufc)
        gs = (gsa, gsb, gsc)
        ss = (ssa, ssb, ssc)

        def g(j, k):
            pltpu.async_copy(src_sh.at[gidx_v.at[j]], bufs[k], gs[k])

        def gwait(j, k):
            pltpu.make_async_copy(src_sh.at[gidx_v.at[j]], bufs[k],
                                  gs[k]).wait()

        def sct(j, k):
            pltpu.async_copy(bufs[k], dst_sh.at[sidx_v.at[j]], ss[k],
                             add=True)

        def swait(j, k):
            pltpu.make_async_copy(bufs[k], dst_sh.at[sidx_v.at[j]],
                                  ss[k]).wait()

        g(0, 0)
        g(1, 1)
        gwait(0, 0)
        sct(0, 0)
        g(2, 2)

        def step(j, carry):
            for k in range(3):
                @pl.when(j % 3 == k)
                def _():
                    gwait(j, k)
                    sct(j, k)
                    swait(j - 1, (k + 2) % 3)
                    g(j + 2, (k + 2) % 3)
            return carry

        lax.fori_loop(1, NCHUNK - 1, step, 0)
        j = NCHUNK - 1
        gwait(j, j % 3)
        sct(j, j % 3)
        swait(j - 1, (j - 1) % 3)
        gwait(NCHUNK, NCHUNK % 3)
        swait(j, j % 3)

    sweep(ya_sh, xe_sh, nidx_v, hidx_v)
    plsc.subcore_barrier()

    def scale(i, carry):
        r = i // (W // L)
        col = (i % (W // L)) * L
        bufa[r, pl.ds(col, L)] = bufa[r, pl.ds(col, L)] * bufb[r, pl.ds(col, L)]
        return carry

    for k in range(RPE // ZB):
        e0 = s * RPE + k * ZB
        pltpu.sync_copy(xe_sh.at[pl.ds(e0, ZB)], ba)
        pltpu.sync_copy(dem_hbm.at[pl.ds(e0, ZB)], bb)
        lax.fori_loop(0, ZB * (W // L), scale, 0)
        pltpu.sync_copy(ba, xe_sh.at[pl.ds(e0, ZB)])

    pltpu.sync_copy(zeros_hbm, ba)
    for k in range(RPN // ZB):
        pltpu.sync_copy(ba, ya_sh.at[pl.ds(s * RPN + k * ZB, ZB)])
    plsc.subcore_barrier()

    sweep(xe_sh, ya_sh, hidx_v, nidx_v)
    plsc.subcore_barrier()

    for k in range(RPN // ZB):
        n0 = s * RPN + k * ZB
        pltpu.sync_copy(ya_sh.at[pl.ds(n0, ZB)], ba)
        pltpu.sync_copy(ba, out_hbm.at[c, pl.ds(n0, ZB)])


def _main_call(ystack, demat, nidx, hidx):
    zeros2 = jnp.zeros((ZB, W), jnp.float32)
    return pl.kernel(
        _main_body,
        out_type=jax.ShapeDtypeStruct((NC, N_PAD, W), jnp.float32),
        mesh=_mesh(),
        compiler_params=_sc_params,
        scratch_types=[
            pltpu.VMEM((TOTCH, CH), jnp.int32),
            pltpu.VMEM((TOTCH, CH), jnp.int32),
            pltpu.VMEM((CH, W), jnp.float32),
            pltpu.VMEM((CH, W), jnp.float32),
            pltpu.VMEM((CH, W), jnp.float32),
            pltpu.VMEM_SHARED((E_PAD, W), jnp.float32),
            pltpu.VMEM_SHARED((N_PAD, W), jnp.float32),
            pltpu.SemaphoreType.DMA,
            pltpu.SemaphoreType.DMA,
            pltpu.SemaphoreType.DMA,
            pltpu.SemaphoreType.DMA,
            pltpu.SemaphoreType.DMA,
            pltpu.SemaphoreType.DMA,
        ],
    )(ystack, demat, nidx, hidx, zeros2)


def _tc1_body(feat_ref, win_ref, bin_ref, w2_ref, b2_ref, dvp_ref, dep_ref,
              xt_ref, y_ref, dem_ref):
    x = feat_ref[...]
    h = lax.dot_general(x, win_ref[...], (((1,), (1,)), ((), ())),
                        preferred_element_type=jnp.float32) + bin_ref[...]
    h = jnp.where(h >= 0, h, 0.2 * h)
    xt = lax.dot_general(h, w2_ref[...], (((1,), (1,)), ((), ())),
                         preferred_element_type=jnp.float32) + b2_ref[...]
    dv = jnp.sum(dvp_ref[...], axis=0)
    dvis = jnp.where(dv > 0, lax.rsqrt(dv), 0.0)
    xt_ref[...] = xt
    y = xt * dvis[:, None]
    y_ref[0] = y[:, :W]
    y_ref[1] = y[:, W:]
    de = jnp.sum(dep_ref[...], axis=0)
    deinv = jnp.where(de > 0, 1.0 / de, 0.0)
    dem_ref[...] = jnp.broadcast_to(deinv[:, None], (E_PAD // 10, W))


def _tc1_call(feat_p, w_in, b_in, w2, b2, dvp, dep):
    blk = 1024
    eblk = E_PAD // 10
    grid = N_PAD // blk
    return pl.pallas_call(
        _tc1_body,
        grid=(grid,),
        in_specs=[
            pl.BlockSpec((blk, D), lambda i: (i, 0)),
            pl.BlockSpec((D, D), lambda i: (0, 0)),
            pl.BlockSpec((1, D), lambda i: (0, 0)),
            pl.BlockSpec((D, D), lambda i: (0, 0)),
            pl.BlockSpec((1, D), lambda i: (0, 0)),
            pl.BlockSpec((NW, blk), lambda i: (0, i)),
            pl.BlockSpec((NW, eblk), lambda i: (0, i)),
        ],
        out_specs=[pl.BlockSpec((blk, D), lambda i: (i, 0)),
                   pl.BlockSpec((NC, blk, W), lambda i: (0, i, 0)),
                   pl.BlockSpec((eblk, W), lambda i: (i, 0))],
        out_shape=[jax.ShapeDtypeStruct((N_PAD, D), jnp.float32),
                   jax.ShapeDtypeStruct((NC, N_PAD, W), jnp.float32),
                   jax.ShapeDtypeStruct((E_PAD, W), jnp.float32)],
    )(feat_p, w_in, b_in, w2, b2, dvp, dep)


def _tc3_body(pn_ref, dvp_ref, xt_ref, o_ref):
    p = jnp.concatenate([pn_ref[0], pn_ref[1]], axis=1)
    dv = jnp.sum(dvp_ref[...], axis=0)
    dvis = jnp.where(dv > 0, lax.rsqrt(dv), 0.0)
    o_ref[...] = jnp.maximum(p * dvis[:, None] + xt_ref[...], 0.0)


def _tc3_call(pn, dvp, xt):
    blk = 1024
    grid = N_PAD // blk
    return pl.pallas_call(
        _tc3_body,
        grid=(grid,),
        in_specs=[
            pl.BlockSpec((NC, blk, W), lambda i: (0, i, 0)),
            pl.BlockSpec((NW, blk), lambda i: (0, i)),
            pl.BlockSpec((blk, D), lambda i: (i, 0)),
        ],
        out_specs=pl.BlockSpec((blk, D), lambda i: (i, 0)),
        out_shape=jax.ShapeDtypeStruct((N_PAD, D), jnp.float32),
    )(pn, dvp, xt)[:N_NODES]


def _pad_idx(idx, pad_val):
    cols = TOTCH * CH - EPT
    return jnp.concatenate(
        [idx.reshape(NS, EPT),
         jnp.full((NS, cols), pad_val, jnp.int32)],
        axis=1).reshape(NS, TOTCH, CH)


def kernel(feat, node_idx, hedge_idx, W_in, b_in, W1, b1, W2, b2):
    f32 = jnp.float32
    feat_p = jnp.zeros((N_PAD, D), f32).at[:N_NODES, :].set(feat)
    node_p = _pad_idx(node_idx, PAD_NODE)
    hedge_p = _pad_idx(hedge_idx, PAD_HEDGE)
    zeros1 = jnp.zeros((N_PAD // L, L), f32)

    dvp, dep = _hist_call(node_p, hedge_p, zeros1)
    dvp = dvp.reshape(NW, N_PAD)
    dep = dep.reshape(NW, E_PAD)
    xt, ystack, demat = _tc1_call(feat_p, W_in, b_in.reshape(1, D), W2,
                                  b2.reshape(1, D), dvp, dep)
    pn = _main_call(ystack, demat, node_p, hedge_p)
    return _tc3_call(pn, dvp, xt)


# speedup vs baseline: 1.2450x; 1.0006x over previous
import functools

import jax
import jax.numpy as jnp
from jax import lax
from jax.experimental import pallas as pl
from jax.experimental.pallas import tpu as pltpu
from jax.experimental.pallas import tpu_sc as plsc

N_NODES = 10000
N_HEDGES = 5000
NNZ = 320000
D = 128
W = 64

NC = 2
NS = 16
NW = NC * NS
L = 16

CH = 128
EPT = NNZ // NS
NCHUNK = 159
TOTCH = 160
NCH_H = 80
N_PAD = 10240
E_PAD = 5120
PAD_NODE = N_PAD - 1
PAD_HEDGE = E_PAD - 1
RPE = E_PAD // NS
RPN = N_PAD // NS

_mesh = functools.partial(
    plsc.VectorSubcoreMesh, core_axis_name="c", subcore_axis_name="s",
    num_cores=NC, num_subcores=NS)
_sc_params = pltpu.CompilerParams(needs_layout_passes=False,
                                  use_tc_tiling_on_sc=False)


def _hist_body(node_hbm, hedge_hbm, zeros_hbm, dvp_hbm, dep_hbm,
               nidx_v, hidx_v, histn_v, histe_v):
    c = lax.axis_index("c")
    s = lax.axis_index("s")
    w = s * NC + c
    pltpu.sync_copy(zeros_hbm, histn_v)
    pltpu.sync_copy(zeros_hbm.at[pl.ds(0, E_PAD // L)], histe_v)
    pltpu.sync_copy(node_hbm.at[s, pl.ds(c * NCH_H, NCH_H)], nidx_v)
    pltpu.sync_copy(hedge_hbm.at[s, pl.ds(c * NCH_H, NCH_H)], hidx_v)
    ones = jnp.full((L,), 1.0, jnp.float32)

    def body(i, carry):
        j = i // (CH // L)
        col = (i % (CH // L)) * L
        nv = nidx_v[j, pl.ds(col, L)]
        plsc.addupdate_scatter(histn_v, [nv >> 4, nv & 15], ones)
        hv = hidx_v[j, pl.ds(col, L)]
        plsc.addupdate_scatter(histe_v, [hv >> 4, hv & 15], ones)
        return carry

    lax.fori_loop(0, NCH_H * (CH // L), body, 0)
    pltpu.sync_copy(histn_v, dvp_hbm.at[w])
    pltpu.sync_copy(histe_v, dep_hbm.at[w])


def _hist_call(node_p, hedge_p, zeros1):
    return pl.kernel(
        _hist_body,
        out_type=(jax.ShapeDtypeStruct((NW, N_PAD // L, L), jnp.float32),
                  jax.ShapeDtypeStruct((NW, E_PAD // L, L), jnp.float32)),
        mesh=_mesh(),
        compiler_params=_sc_params,
        scratch_types=[
            pltpu.VMEM((NCH_H, CH), jnp.int32),
            pltpu.VMEM((NCH_H, CH), jnp.int32),
            pltpu.VMEM((N_PAD // L, L), jnp.float32),
            pltpu.VMEM((E_PAD // L, L), jnp.float32),
        ],
    )(node_p, hedge_p, zeros1)


ZB = 64


def _main_body(y_hbm, dem_hbm, nidx_hbm, hidx_hbm, zeros_hbm, out_hbm,
               nidx_v, hidx_v, bufa, bufb, bufc, xe_sh, ya_sh,
               gsa, gsb, gsc, ssa, ssb, ssc):
    c = lax.axis_index("c")
    s = lax.axis_index("s")
    ba = bufa.at[pl.ds(0, ZB)]
    bb = bufb.at[pl.ds(0, ZB)]

    pltpu.sync_copy(nidx_hbm.at[s], nidx_v)
    pltpu.sync_copy(hidx_hbm.at[s], hidx_v)
    for k in range(RPN // ZB):
        r0 = s * RPN + k * ZB
        pltpu.sync_copy(y_hbm.at[c, pl.ds(r0, ZB)], ya_sh.at[pl.ds(r0, ZB)])
    pltpu.sync_copy(zeros_hbm, ba)
    for k in range(RPE // ZB):
        pltpu.sync_copy(ba, xe_sh.at[pl.ds(s * RPE + k * ZB, ZB)])
    plsc.subcore_barrier()

    def sweep(src_sh, dst_sh, gidx_v, sidx_v):
        bufs = (bufa, bufb, bufc)
        gs = (gsa, gsb, gsc)
        ss = (ssa, ssb, ssc)

        def g(j, k):
            pltpu.async_copy(src_sh.at[gidx_v.at[j]], bufs[k], gs[k])

        def gwait(j, k):
            pltpu.make_async_copy(src_sh.at[gidx_v.at[j]], bufs[k],
                                  gs[k]).wait()

        def sct(j, k):
            pltpu.async_copy(bufs[k], dst_sh.at[sidx_v.at[j]], ss[k],
                             add=True)

        def swait(j, k):
            pltpu.make_async_copy(bufs[k], dst_sh.at[sidx_v.at[j]],
                                  ss[k]).wait()

        g(0, 0)
        g(1, 1)
        gwait(0, 0)
        sct(0, 0)
        g(2, 2)

        def step(j, carry):
            for k in range(3):
                @pl.when(j % 3 == k)
                def _():
                    gwait(j, k)
                    sct(j, k)
                    swait(j - 1, (k + 2) % 3)
                    g(j + 2, (k + 2) % 3)
            return carry

        lax.fori_loop(1, NCHUNK - 1, step, 0)
        j = NCHUNK - 1
        gwait(j, j % 3)
        sct(j, j % 3)
        swait(j - 1, (j - 1) % 3)
        gwait(NCHUNK, NCHUNK % 3)
        swait(j, j % 3)

    sweep(ya_sh, xe_sh, nidx_v, hidx_v)
    plsc.subcore_barrier()

    def scale(i, carry):
        r = i // (W // L)
        col = (i % (W // L)) * L
        bufa[r, pl.ds(col, L)] = bufa[r, pl.ds(col, L)] * bufb[r, pl.ds(col, L)]
        return carry

    for k in range(RPE // ZB):
        e0 = s * RPE + k * ZB
        pltpu.sync_copy(xe_sh.at[pl.ds(e0, ZB)], ba)
        pltpu.sync_copy(dem_hbm.at[pl.ds(e0, ZB)], bb)
        lax.fori_loop(0, ZB * (W // L), scale, 0)
        pltpu.sync_copy(ba, xe_sh.at[pl.ds(e0, ZB)])

    pltpu.sync_copy(zeros_hbm, ba)
    for k in range(RPN // ZB):
        pltpu.sync_copy(ba, ya_sh.at[pl.ds(s * RPN + k * ZB, ZB)])
    plsc.subcore_barrier()

    sweep(xe_sh, ya_sh, hidx_v, nidx_v)
    plsc.subcore_barrier()

    for k in range(RPN // ZB):
        n0 = s * RPN + k * ZB
        pltpu.sync_copy(ya_sh.at[pl.ds(n0, ZB)], ba)
        pltpu.sync_copy(ba, out_hbm.at[c, pl.ds(n0, ZB)])


def _main_call(ystack, demat, nidx, hidx):
    zeros2 = jnp.zeros((ZB, W), jnp.float32)
    return pl.kernel(
        _main_body,
        out_type=jax.ShapeDtypeStruct((NC, N_PAD, W), jnp.float32),
        mesh=_mesh(),
        compiler_params=_sc_params,
        scratch_types=[
            pltpu.VMEM((TOTCH, CH), jnp.int32),
            pltpu.VMEM((TOTCH, CH), jnp.int32),
            pltpu.VMEM((CH, W), jnp.float32),
            pltpu.VMEM((CH, W), jnp.float32),
            pltpu.VMEM((CH, W), jnp.float32),
            pltpu.VMEM_SHARED((E_PAD, W), jnp.float32),
            pltpu.VMEM_SHARED((N_PAD, W), jnp.float32),
            pltpu.SemaphoreType.DMA,
            pltpu.SemaphoreType.DMA,
            pltpu.SemaphoreType.DMA,
            pltpu.SemaphoreType.DMA,
            pltpu.SemaphoreType.DMA,
            pltpu.SemaphoreType.DMA,
        ],
    )(ystack, demat, nidx, hidx, zeros2)


def _tc1a_body(feat_ref, win_ref, bin_ref, w2_ref, b2_ref, xt_ref):
    x = feat_ref[...]
    h = lax.dot_general(x, win_ref[...], (((1,), (1,)), ((), ())),
                        preferred_element_type=jnp.float32) + bin_ref[...]
    h = jnp.where(h >= 0, h, 0.2 * h)
    xt_ref[...] = lax.dot_general(h, w2_ref[...], (((1,), (1,)), ((), ())),
                                  preferred_element_type=jnp.float32) + b2_ref[...]


def _tc1a_call(feat_p, w_in, b_in, w2, b2):
    blk = 1024
    grid = N_PAD // blk
    return pl.pallas_call(
        _tc1a_body,
        grid=(grid,),
        in_specs=[
            pl.BlockSpec((blk, D), lambda i: (i, 0)),
            pl.BlockSpec((D, D), lambda i: (0, 0)),
            pl.BlockSpec((1, D), lambda i: (0, 0)),
            pl.BlockSpec((D, D), lambda i: (0, 0)),
            pl.BlockSpec((1, D), lambda i: (0, 0)),
        ],
        out_specs=pl.BlockSpec((blk, D), lambda i: (i, 0)),
        out_shape=jax.ShapeDtypeStruct((N_PAD, D), jnp.float32),
    )(feat_p, w_in, b_in, w2, b2)


def _tc1b_body(xt_ref, dvp_ref, dep_ref, y_ref, dem_ref):
    xt = xt_ref[...]
    dv = jnp.sum(dvp_ref[...], axis=0)
    dvis = jnp.where(dv > 0, lax.rsqrt(dv), 0.0)
    y = xt * dvis[:, None]
    y_ref[0] = y[:, :W]
    y_ref[1] = y[:, W:]
    de = jnp.sum(dep_ref[...], axis=0)
    deinv = jnp.where(de > 0, 1.0 / de, 0.0)
    dem_ref[...] = jnp.broadcast_to(deinv[:, None], dem_ref.shape)


def _tc1b_call(xt, dvp, dep):
    blk = 1024
    eblk = E_PAD // 10
    grid = N_PAD // blk
    return pl.pallas_call(
        _tc1b_body,
        grid=(grid,),
        in_specs=[
            pl.BlockSpec((blk, D), lambda i: (i, 0)),
            pl.BlockSpec((NW, blk), lambda i: (0, i)),
            pl.BlockSpec((NW, eblk), lambda i: (0, i)),
        ],
        out_specs=[pl.BlockSpec((NC, blk, W), lambda i: (0, i, 0)),
                   pl.BlockSpec((eblk, W), lambda i: (i, 0))],
        out_shape=[jax.ShapeDtypeStruct((NC, N_PAD, W), jnp.float32),
                   jax.ShapeDtypeStruct((E_PAD, W), jnp.float32)],
    )(xt, dvp, dep)


def _tc3_body(pn_ref, dvp_ref, xt_ref, o_ref):
    p = jnp.concatenate([pn_ref[0], pn_ref[1]], axis=1)
    dv = jnp.sum(dvp_ref[...], axis=0)
    dvis = jnp.where(dv > 0, lax.rsqrt(dv), 0.0)
    o_ref[...] = jnp.maximum(p * dvis[:, None] + xt_ref[...], 0.0)


def _tc3_call(pn, dvp, xt):
    blk = 1024
    grid = N_PAD // blk
    return pl.pallas_call(
        _tc3_body,
        grid=(grid,),
        in_specs=[
            pl.BlockSpec((NC, blk, W), lambda i: (0, i, 0)),
            pl.BlockSpec((NW, blk), lambda i: (0, i)),
            pl.BlockSpec((blk, D), lambda i: (i, 0)),
        ],
        out_specs=pl.BlockSpec((blk, D), lambda i: (i, 0)),
        out_shape=jax.ShapeDtypeStruct((N_PAD, D), jnp.float32),
    )(pn, dvp, xt)[:N_NODES]


def _pad_idx(idx, pad_val):
    cols = TOTCH * CH - EPT
    return jnp.concatenate(
        [idx.reshape(NS, EPT),
         jnp.full((NS, cols), pad_val, jnp.int32)],
        axis=1).reshape(NS, TOTCH, CH)


def kernel(feat, node_idx, hedge_idx, W_in, b_in, W1, b1, W2, b2):
    f32 = jnp.float32
    feat_p = jnp.zeros((N_PAD, D), f32).at[:N_NODES, :].set(feat)
    node_p = _pad_idx(node_idx, PAD_NODE)
    hedge_p = _pad_idx(hedge_idx, PAD_HEDGE)
    zeros1 = jnp.zeros((N_PAD // L, L), f32)

    dvp, dep = _hist_call(node_p, hedge_p, zeros1)
    xt = _tc1a_call(feat_p, W_in, b_in.reshape(1, D), W2, b2.reshape(1, D))
    dvp = dvp.reshape(NW, N_PAD)
    dep = dep.reshape(NW, E_PAD)
    ystack, demat = _tc1b_call(xt, dvp, dep)
    pn = _main_call(ystack, demat, node_p, hedge_p)
    return _tc3_call(pn, dvp, xt)


# speedup vs baseline: 1.2792x; 1.0275x over previous
import functools

import jax
import jax.numpy as jnp
from jax import lax
from jax.experimental import pallas as pl
from jax.experimental.pallas import tpu as pltpu
from jax.experimental.pallas import tpu_sc as plsc

N_NODES = 10000
N_HEDGES = 5000
NNZ = 320000
D = 128
W = 64

NC = 2
NS = 16
NW = NC * NS
L = 16

CH = 128
EPT = NNZ // NS
NCHUNK = 159
TOTCH = 160
NCH_H = 80
N_PAD = 10240
E_PAD = 5120
PAD_NODE = N_PAD - 1
PAD_HEDGE = E_PAD - 1
RPE = E_PAD // NS
RPN = N_PAD // NS

_mesh = functools.partial(
    plsc.VectorSubcoreMesh, core_axis_name="c", subcore_axis_name="s",
    num_cores=NC, num_subcores=NS)
_sc_params = pltpu.CompilerParams(needs_layout_passes=False,
                                  use_tc_tiling_on_sc=False)


def _hist_body(node_hbm, hedge_hbm, zeros_hbm, dvp_hbm, dep_hbm,
               nidx_v, hidx_v, histn_v, histe_v):
    c = lax.axis_index("c")
    s = lax.axis_index("s")
    w = s * NC + c
    pltpu.sync_copy(zeros_hbm, histn_v)
    pltpu.sync_copy(zeros_hbm.at[pl.ds(0, E_PAD // L)], histe_v)
    pltpu.sync_copy(node_hbm.at[s, pl.ds(c * NCH_H, NCH_H)], nidx_v)
    pltpu.sync_copy(hedge_hbm.at[s, pl.ds(c * NCH_H, NCH_H)], hidx_v)
    ones = jnp.full((L,), 1.0, jnp.float32)

    def body(i, carry):
        j = i // (CH // L)
        col = (i % (CH // L)) * L
        nv = nidx_v[j, pl.ds(col, L)]
        plsc.addupdate_scatter(histn_v, [nv >> 4, nv & 15], ones)
        hv = hidx_v[j, pl.ds(col, L)]
        plsc.addupdate_scatter(histe_v, [hv >> 4, hv & 15], ones)
        return carry

    lax.fori_loop(0, NCH_H * (CH // L), body, 0)
    pltpu.sync_copy(histn_v, dvp_hbm.at[w])
    pltpu.sync_copy(histe_v, dep_hbm.at[w])


def _hist_call(node_p, hedge_p, zeros1):
    return pl.kernel(
        _hist_body,
        out_type=(jax.ShapeDtypeStruct((NW, N_PAD // L, L), jnp.float32),
                  jax.ShapeDtypeStruct((NW, E_PAD // L, L), jnp.float32)),
        mesh=_mesh(),
        compiler_params=_sc_params,
        scratch_types=[
            pltpu.VMEM((NCH_H, CH), jnp.int32),
            pltpu.VMEM((NCH_H, CH), jnp.int32),
            pltpu.VMEM((N_PAD // L, L), jnp.float32),
            pltpu.VMEM((E_PAD // L, L), jnp.float32),
        ],
    )(node_p, hedge_p, zeros1)


ZB = 64


def _main_body(y_hbm, dem_hbm, nidx_hbm, hidx_hbm, zeros_hbm, out_hbm,
               nidx_v, hidx_v, bufa, bufb, bufc, xe_sh, ya_sh,
               gsa, gsb, gsc, ssa, ssb, ssc):
    c = lax.axis_index("c")
    s = lax.axis_index("s")
    ba = bufa.at[pl.ds(0, ZB)]
    bb = bufb.at[pl.ds(0, ZB)]

    pltpu.async_copy(nidx_hbm.at[s], nidx_v, gsa)
    pltpu.async_copy(hidx_hbm.at[s], hidx_v, gsb)
    for k in range(RPN // ZB):
        r0 = s * RPN + k * ZB
        pltpu.async_copy(y_hbm.at[c, pl.ds(r0, ZB)], ya_sh.at[pl.ds(r0, ZB)],
                         gsc)
    pltpu.sync_copy(zeros_hbm, ba)
    for k in range(RPE // ZB):
        pltpu.async_copy(ba, xe_sh.at[pl.ds(s * RPE + k * ZB, ZB)], ssa)
    for k in range(RPN // ZB):
        r0 = s * RPN + k * ZB
        pltpu.make_async_copy(y_hbm.at[c, pl.ds(r0, ZB)],
                              ya_sh.at[pl.ds(r0, ZB)], gsc).wait()
    for k in range(RPE // ZB):
        pltpu.make_async_copy(ba, xe_sh.at[pl.ds(s * RPE + k * ZB, ZB)],
                              ssa).wait()
    pltpu.make_async_copy(nidx_hbm.at[s], nidx_v, gsa).wait()
    pltpu.make_async_copy(hidx_hbm.at[s], hidx_v, gsb).wait()
    plsc.subcore_barrier()

    def sweep(src_sh, dst_sh, gidx_v, sidx_v):
        bufs = (bufa, bufb, bufc)
        gs = (gsa, gsb, gsc)
        ss = (ssa, ssb, ssc)

        def g(j, k):
            pltpu.async_copy(src_sh.at[gidx_v.at[j]], bufs[k], gs[k])

        def gwait(j, k):
            pltpu.make_async_copy(src_sh.at[gidx_v.at[j]], bufs[k],
                                  gs[k]).wait()

        def sct(j, k):
            pltpu.async_copy(bufs[k], dst_sh.at[sidx_v.at[j]], ss[k],
                             add=True)

        def swait(j, k):
            pltpu.make_async_copy(bufs[k], dst_sh.at[sidx_v.at[j]],
                                  ss[k]).wait()

        g(0, 0)
        g(1, 1)
        gwait(0, 0)
        sct(0, 0)
        g(2, 2)

        def step(j, carry):
            for k in range(3):
                @pl.when(j % 3 == k)
                def _():
                    gwait(j, k)
                    sct(j, k)
                    swait(j - 1, (k + 2) % 3)
                    g(j + 2, (k + 2) % 3)
            return carry

        lax.fori_loop(1, NCHUNK - 1, step, 0)
        j = NCHUNK - 1
        gwait(j, j % 3)
        sct(j, j % 3)
        swait(j - 1, (j - 1) % 3)
        gwait(NCHUNK, NCHUNK % 3)
        swait(j, j % 3)

    sweep(ya_sh, xe_sh, nidx_v, hidx_v)
    plsc.subcore_barrier()

    def scale(i, carry):
        r = i // (W // L)
        col = (i % (W // L)) * L
        bufa[r, pl.ds(col, L)] = bufa[r, pl.ds(col, L)] * bufb[r, pl.ds(col, L)]
        return carry

    for k in range(RPE // ZB):
        e0 = s * RPE + k * ZB
        pltpu.sync_copy(xe_sh.at[pl.ds(e0, ZB)], ba)
        pltpu.sync_copy(dem_hbm.at[pl.ds(e0, ZB)], bb)
        lax.fori_loop(0, ZB * (W // L), scale, 0)
        pltpu.sync_copy(ba, xe_sh.at[pl.ds(e0, ZB)])

    pltpu.sync_copy(zeros_hbm, ba)
    for k in range(RPN // ZB):
        pltpu.async_copy(ba, ya_sh.at[pl.ds(s * RPN + k * ZB, ZB)], ssa)
    for k in range(RPN // ZB):
        pltpu.make_async_copy(ba, ya_sh.at[pl.ds(s * RPN + k * ZB, ZB)],
                              ssa).wait()
    plsc.subcore_barrier()

    sweep(xe_sh, ya_sh, hidx_v, nidx_v)
    plsc.subcore_barrier()

    for k in range(RPN // ZB):
        n0 = s * RPN + k * ZB
        stg = (bufa, bufb)[k % 2].at[pl.ds(0, ZB)]
        sem = (gsa, gsb)[k % 2]
        if k >= 2:
            p0 = s * RPN + (k - 2) * ZB
            pltpu.make_async_copy(stg, out_hbm.at[c, pl.ds(p0, ZB)],
                                  sem).wait()
        pltpu.sync_copy(ya_sh.at[pl.ds(n0, ZB)], stg)
        pltpu.async_copy(stg, out_hbm.at[c, pl.ds(n0, ZB)], sem)
    for k in range(RPN // ZB - 2, RPN // ZB):
        n0 = s * RPN + k * ZB
        stg = (bufa, bufb)[k % 2].at[pl.ds(0, ZB)]
        sem = (gsa, gsb)[k % 2]
        pltpu.make_async_copy(stg, out_hbm.at[c, pl.ds(n0, ZB)], sem).wait()


def _main_call(ystack, demat, nidx, hidx):
    zeros2 = jnp.zeros((ZB, W), jnp.float32)
    return pl.kernel(
        _main_body,
        out_type=jax.ShapeDtypeStruct((NC, N_PAD, W), jnp.float32),
        mesh=_mesh(),
        compiler_params=_sc_params,
        scratch_types=[
            pltpu.VMEM((TOTCH, CH), jnp.int32),
            pltpu.VMEM((TOTCH, CH), jnp.int32),
            pltpu.VMEM((CH, W), jnp.float32),
            pltpu.VMEM((CH, W), jnp.float32),
            pltpu.VMEM((CH, W), jnp.float32),
            pltpu.VMEM_SHARED((E_PAD, W), jnp.float32),
            pltpu.VMEM_SHARED((N_PAD, W), jnp.float32),
            pltpu.SemaphoreType.DMA,
            pltpu.SemaphoreType.DMA,
            pltpu.SemaphoreType.DMA,
            pltpu.SemaphoreType.DMA,
            pltpu.SemaphoreType.DMA,
            pltpu.SemaphoreType.DMA,
        ],
    )(ystack, demat, nidx, hidx, zeros2)


def _tc1_body(feat_ref, win_ref, bin_ref, w2_ref, b2_ref, dvp_ref, dep_ref,
              xt_ref, y_ref, dem_ref):
    x = feat_ref[...]
    h = lax.dot_general(x, win_ref[...], (((1,), (1,)), ((), ())),
                        preferred_element_type=jnp.float32) + bin_ref[...]
    h = jnp.where(h >= 0, h, 0.2 * h)
    xt = lax.dot_general(h, w2_ref[...], (((1,), (1,)), ((), ())),
                         preferred_element_type=jnp.float32) + b2_ref[...]
    dv = jnp.sum(dvp_ref[...], axis=0)
    dvis = jnp.where(dv > 0, lax.rsqrt(dv), 0.0)
    xt_ref[...] = xt
    y = xt * dvis[:, None]
    y_ref[0] = y[:, :W]
    y_ref[1] = y[:, W:]
    de = jnp.sum(dep_ref[...], axis=0)
    deinv = jnp.where(de > 0, 1.0 / de, 0.0)
    dem_ref[...] = jnp.broadcast_to(deinv[:, None], (E_PAD // 10, W))


def _tc1_call(feat_p, w_in, b_in, w2, b2, dvp, dep):
    blk = 1024
    eblk = E_PAD // 10
    grid = N_PAD // blk
    return pl.pallas_call(
        _tc1_body,
        grid=(grid,),
        in_specs=[
            pl.BlockSpec((blk, D), lambda i: (i, 0)),
            pl.BlockSpec((D, D), lambda i: (0, 0)),
            pl.BlockSpec((1, D), lambda i: (0, 0)),
            pl.BlockSpec((D, D), lambda i: (0, 0)),
            pl.BlockSpec((1, D), lambda i: (0, 0)),
            pl.BlockSpec((NW, blk), lambda i: (0, i)),
            pl.BlockSpec((NW, eblk), lambda i: (0, i)),
        ],
        out_specs=[pl.BlockSpec((blk, D), lambda i: (i, 0)),
                   pl.BlockSpec((NC, blk, W), lambda i: (0, i, 0)),
                   pl.BlockSpec((eblk, W), lambda i: (i, 0))],
        out_shape=[jax.ShapeDtypeStruct((N_PAD, D), jnp.float32),
                   jax.ShapeDtypeStruct((NC, N_PAD, W), jnp.float32),
                   jax.ShapeDtypeStruct((E_PAD, W), jnp.float32)],
    )(feat_p, w_in, b_in, w2, b2, dvp, dep)


def _tc3_body(pn_ref, dvp_ref, xt_ref, o_ref):
    p = jnp.concatenate([pn_ref[0], pn_ref[1]], axis=1)
    dv = jnp.sum(dvp_ref[...], axis=0)
    dvis = jnp.where(dv > 0, lax.rsqrt(dv), 0.0)
    o_ref[...] = jnp.maximum(p * dvis[:, None] + xt_ref[...], 0.0)


def _tc3_call(pn, dvp, xt):
    blk = 1024
    grid = N_PAD // blk
    return pl.pallas_call(
        _tc3_body,
        grid=(grid,),
        in_specs=[
            pl.BlockSpec((NC, blk, W), lambda i: (0, i, 0)),
            pl.BlockSpec((NW, blk), lambda i: (0, i)),
            pl.BlockSpec((blk, D), lambda i: (i, 0)),
        ],
        out_specs=pl.BlockSpec((blk, D), lambda i: (i, 0)),
        out_shape=jax.ShapeDtypeStruct((N_PAD, D), jnp.float32),
    )(pn, dvp, xt)[:N_NODES]


def _pad_idx(idx, pad_val):
    cols = TOTCH * CH - EPT
    return jnp.concatenate(
        [idx.reshape(NS, EPT),
         jnp.full((NS, cols), pad_val, jnp.int32)],
        axis=1).reshape(NS, TOTCH, CH)


def kernel(feat, node_idx, hedge_idx, W_in, b_in, W1, b1, W2, b2):
    f32 = jnp.float32
    feat_p = jnp.zeros((N_PAD, D), f32).at[:N_NODES, :].set(feat)
    node_p = _pad_idx(node_idx, PAD_NODE)
    hedge_p = _pad_idx(hedge_idx, PAD_HEDGE)
    zeros1 = jnp.zeros((N_PAD // L, L), f32)

    dvp, dep = _hist_call(node_p, hedge_p, zeros1)
    dvp = dvp.reshape(NW, N_PAD)
    dep = dep.reshape(NW, E_PAD)
    xt, ystack, demat = _tc1_call(feat_p, W_in, b_in.reshape(1, D), W2,
                                  b2.reshape(1, D), dvp, dep)
    pn = _main_call(ystack, demat, node_p, hedge_p)
    return _tc3_call(pn, dvp, xt)


# speedup vs baseline: 1.2952x; 1.0125x over previous
import functools

import jax
import jax.numpy as jnp
from jax import lax
from jax.experimental import pallas as pl
from jax.experimental.pallas import tpu as pltpu
from jax.experimental.pallas import tpu_sc as plsc

N_NODES = 10000
N_HEDGES = 5000
NNZ = 320000
D = 128
W = 64

NC = 2
NS = 16
NW = NC * NS
L = 16

CH = 128
EPT = NNZ // NS
NCHUNK = 159
TOTCH = 160
NCH_H = 80
N_PAD = 10240
E_PAD = 5120
PAD_NODE = N_PAD - 1
PAD_HEDGE = E_PAD - 1
RPE = E_PAD // NS
RPN = N_PAD // NS

_mesh = functools.partial(
    plsc.VectorSubcoreMesh, core_axis_name="c", subcore_axis_name="s",
    num_cores=NC, num_subcores=NS)
_sc_params = pltpu.CompilerParams(needs_layout_passes=False,
                                  use_tc_tiling_on_sc=False)


def _hist_body(node_hbm, hedge_hbm, zeros_hbm, dvp_hbm, dep_hbm,
               nidx_v, hidx_v, histn_v, histe_v):
    c = lax.axis_index("c")
    s = lax.axis_index("s")
    w = s * NC + c
    pltpu.sync_copy(zeros_hbm, histn_v)
    pltpu.sync_copy(zeros_hbm.at[pl.ds(0, E_PAD // L)], histe_v)
    pltpu.sync_copy(node_hbm.at[s, pl.ds(c * NCH_H, NCH_H)], nidx_v)
    pltpu.sync_copy(hedge_hbm.at[s, pl.ds(c * NCH_H, NCH_H)], hidx_v)
    ones = jnp.full((L,), 1.0, jnp.float32)

    def body(i, carry):
        j = i // (CH // L)
        col = (i % (CH // L)) * L
        nv = nidx_v[j, pl.ds(col, L)]
        plsc.addupdate_scatter(histn_v, [nv >> 4, nv & 15], ones)
        hv = hidx_v[j, pl.ds(col, L)]
        plsc.addupdate_scatter(histe_v, [hv >> 4, hv & 15], ones)
        return carry

    lax.fori_loop(0, NCH_H * (CH // L), body, 0)
    pltpu.sync_copy(histn_v, dvp_hbm.at[w])
    pltpu.sync_copy(histe_v, dep_hbm.at[w])


def _hist_call(node_p, hedge_p, zeros1):
    return pl.kernel(
        _hist_body,
        out_type=(jax.ShapeDtypeStruct((NW, N_PAD // L, L), jnp.float32),
                  jax.ShapeDtypeStruct((NW, E_PAD // L, L), jnp.float32)),
        mesh=_mesh(),
        compiler_params=_sc_params,
        scratch_types=[
            pltpu.VMEM((NCH_H, CH), jnp.int32),
            pltpu.VMEM((NCH_H, CH), jnp.int32),
            pltpu.VMEM((N_PAD // L, L), jnp.float32),
            pltpu.VMEM((E_PAD // L, L), jnp.float32),
        ],
    )(node_p, hedge_p, zeros1)


ZB = 64


def _main_body(y_hbm, dem_hbm, nidx_hbm, hidx_hbm, zeros_hbm, out_hbm,
               nidx_v, hidx_v, bufa, bufb, bufc, xe_sh, ya_sh,
               gsa, gsb, gsc, ssa, ssb, ssc):
    c = lax.axis_index("c")
    s = lax.axis_index("s")
    ba = bufa.at[pl.ds(0, ZB)]
    bb = bufb.at[pl.ds(0, ZB)]

    pltpu.async_copy(nidx_hbm.at[s], nidx_v, gsa)
    pltpu.async_copy(hidx_hbm.at[s], hidx_v, gsb)
    for k in range(RPN // ZB):
        r0 = s * RPN + k * ZB
        pltpu.async_copy(y_hbm.at[c, pl.ds(r0, ZB)], ya_sh.at[pl.ds(r0, ZB)],
                         gsc)
    pltpu.sync_copy(zeros_hbm, ba)
    for k in range(RPE // ZB):
        pltpu.async_copy(ba, xe_sh.at[pl.ds(s * RPE + k * ZB, ZB)], ssa)
    for k in range(RPN // ZB):
        r0 = s * RPN + k * ZB
        pltpu.make_async_copy(y_hbm.at[c, pl.ds(r0, ZB)],
                              ya_sh.at[pl.ds(r0, ZB)], gsc).wait()
    for k in range(RPE // ZB):
        pltpu.make_async_copy(ba, xe_sh.at[pl.ds(s * RPE + k * ZB, ZB)],
                              ssa).wait()
    pltpu.make_async_copy(nidx_hbm.at[s], nidx_v, gsa).wait()
    pltpu.make_async_copy(hidx_hbm.at[s], hidx_v, gsb).wait()
    plsc.subcore_barrier()

    def sweep(src_sh, dst_sh, gidx_v, sidx_v):
        bufs = (bufa, bufb, bufc)
        gs = (gsa, gsb, gsc)
        ss = (ssa, ssb, ssc)

        def g(j, k):
            pltpu.async_copy(src_sh.at[gidx_v.at[j]], bufs[k], gs[k])

        def gwait(j, k):
            pltpu.make_async_copy(src_sh.at[gidx_v.at[j]], bufs[k],
                                  gs[k]).wait()

        def sct(j, k):
            pltpu.async_copy(bufs[k], dst_sh.at[sidx_v.at[j]], ss[k],
                             add=True)

        def swait(j, k):
            pltpu.make_async_copy(bufs[k], dst_sh.at[sidx_v.at[j]],
                                  ss[k]).wait()

        g(0, 0)
        g(1, 1)
        gwait(0, 0)
        sct(0, 0)
        g(2, 2)

        def step(j, carry):
            for k in range(3):
                @pl.when(j % 3 == k)
                def _():
                    gwait(j, k)
                    sct(j, k)
                    swait(j - 1, (k + 2) % 3)
                    g(j + 2, (k + 2) % 3)
            return carry

        lax.fori_loop(1, NCHUNK - 1, step, 0)
        j = NCHUNK - 1
        gwait(j, j % 3)
        sct(j, j % 3)
        swait(j - 1, (j - 1) % 3)
        gwait(NCHUNK, NCHUNK % 3)
        swait(j, j % 3)

    sweep(ya_sh, xe_sh, nidx_v, hidx_v)
    plsc.subcore_barrier()

    def scale(r, carry):
        for col in range(0, W, L):
            bufa[r, pl.ds(col, L)] = (bufa[r, pl.ds(col, L)]
                                      * bufb[r, pl.ds(col, L)])
        return carry

    for k in range(RPE // ZB):
        e0 = s * RPE + k * ZB
        pltpu.sync_copy(xe_sh.at[pl.ds(e0, ZB)], ba)
        pltpu.sync_copy(dem_hbm.at[pl.ds(e0, ZB)], bb)
        lax.fori_loop(0, ZB, scale, 0)
        pltpu.sync_copy(ba, xe_sh.at[pl.ds(e0, ZB)])

    pltpu.sync_copy(zeros_hbm, ba)
    for k in range(RPN // ZB):
        pltpu.async_copy(ba, ya_sh.at[pl.ds(s * RPN + k * ZB, ZB)], ssa)
    for k in range(RPN // ZB):
        pltpu.make_async_copy(ba, ya_sh.at[pl.ds(s * RPN + k * ZB, ZB)],
                              ssa).wait()
    plsc.subcore_barrier()

    sweep(xe_sh, ya_sh, hidx_v, nidx_v)
    plsc.subcore_barrier()

    for k in range(RPN // ZB):
        n0 = s * RPN + k * ZB
        stg = (bufa, bufb)[k % 2].at[pl.ds(0, ZB)]
        sem = (gsa, gsb)[k % 2]
        if k >= 2:
            p0 = s * RPN + (k - 2) * ZB
            pltpu.make_async_copy(stg, out_hbm.at[c, pl.ds(p0, ZB)],
                                  sem).wait()
        pltpu.sync_copy(ya_sh.at[pl.ds(n0, ZB)], stg)
        pltpu.async_copy(stg, out_hbm.at[c, pl.ds(n0, ZB)], sem)
    for k in range(RPN // ZB - 2, RPN // ZB):
        n0 = s * RPN + k * ZB
        stg = (bufa, bufb)[k % 2].at[pl.ds(0, ZB)]
        sem = (gsa, gsb)[k % 2]
        pltpu.make_async_copy(stg, out_hbm.at[c, pl.ds(n0, ZB)], sem).wait()


def _main_call(ystack, demat, nidx, hidx):
    zeros2 = jnp.zeros((ZB, W), jnp.float32)
    return pl.kernel(
        _main_body,
        out_type=jax.ShapeDtypeStruct((NC, N_PAD, W), jnp.float32),
        mesh=_mesh(),
        compiler_params=_sc_params,
        scratch_types=[
            pltpu.VMEM((TOTCH, CH), jnp.int32),
            pltpu.VMEM((TOTCH, CH), jnp.int32),
            pltpu.VMEM((CH, W), jnp.float32),
            pltpu.VMEM((CH, W), jnp.float32),
            pltpu.VMEM((CH, W), jnp.float32),
            pltpu.VMEM_SHARED((E_PAD, W), jnp.float32),
            pltpu.VMEM_SHARED((N_PAD, W), jnp.float32),
            pltpu.SemaphoreType.DMA,
            pltpu.SemaphoreType.DMA,
            pltpu.SemaphoreType.DMA,
            pltpu.SemaphoreType.DMA,
            pltpu.SemaphoreType.DMA,
            pltpu.SemaphoreType.DMA,
        ],
    )(ystack, demat, nidx, hidx, zeros2)


def _tc1_body(feat_ref, win_ref, bin_ref, w2_ref, b2_ref, dvp_ref, dep_ref,
              xt_ref, y_ref, dem_ref):
    x = feat_ref[...]
    h = lax.dot_general(x, win_ref[...], (((1,), (1,)), ((), ())),
                        preferred_element_type=jnp.float32) + bin_ref[...]
    h = jnp.where(h >= 0, h, 0.2 * h)
    xt = lax.dot_general(h, w2_ref[...], (((1,), (1,)), ((), ())),
                         preferred_element_type=jnp.float32) + b2_ref[...]
    dv = jnp.sum(dvp_ref[...], axis=0)
    dvis = jnp.where(dv > 0, lax.rsqrt(dv), 0.0)
    xt_ref[...] = xt
    y = xt * dvis[:, None]
    y_ref[0] = y[:, :W]
    y_ref[1] = y[:, W:]
    de = jnp.sum(dep_ref[...], axis=0)
    deinv = jnp.where(de > 0, 1.0 / de, 0.0)
    dem_ref[...] = jnp.broadcast_to(deinv[:, None], (E_PAD // 10, W))


def _tc1_call(feat_p, w_in, b_in, w2, b2, dvp, dep):
    blk = 1024
    eblk = E_PAD // 10
    grid = N_PAD // blk
    return pl.pallas_call(
        _tc1_body,
        grid=(grid,),
        in_specs=[
            pl.BlockSpec((blk, D), lambda i: (i, 0)),
            pl.BlockSpec((D, D), lambda i: (0, 0)),
            pl.BlockSpec((1, D), lambda i: (0, 0)),
            pl.BlockSpec((D, D), lambda i: (0, 0)),
            pl.BlockSpec((1, D), lambda i: (0, 0)),
            pl.BlockSpec((NW, blk), lambda i: (0, i)),
            pl.BlockSpec((NW, eblk), lambda i: (0, i)),
        ],
        out_specs=[pl.BlockSpec((blk, D), lambda i: (i, 0)),
                   pl.BlockSpec((NC, blk, W), lambda i: (0, i, 0)),
                   pl.BlockSpec((eblk, W), lambda i: (i, 0))],
        out_shape=[jax.ShapeDtypeStruct((N_PAD, D), jnp.float32),
                   jax.ShapeDtypeStruct((NC, N_PAD, W), jnp.float32),
                   jax.ShapeDtypeStruct((E_PAD, W), jnp.float32)],
    )(feat_p, w_in, b_in, w2, b2, dvp, dep)


def _tc3_body(pn_ref, dvp_ref, xt_ref, o_ref):
    p = jnp.concatenate([pn_ref[0], pn_ref[1]], axis=1)
    dv = jnp.sum(dvp_ref[...], axis=0)
    dvis = jnp.where(dv > 0, lax.rsqrt(dv), 0.0)
    o_ref[...] = jnp.maximum(p * dvis[:, None] + xt_ref[...], 0.0)


def _tc3_call(pn, dvp, xt):
    blk = 1024
    grid = N_PAD // blk
    return pl.pallas_call(
        _tc3_body,
        grid=(grid,),
        in_specs=[
            pl.BlockSpec((NC, blk, W), lambda i: (0, i, 0)),
            pl.BlockSpec((NW, blk), lambda i: (0, i)),
            pl.BlockSpec((blk, D), lambda i: (i, 0)),
        ],
        out_specs=pl.BlockSpec((blk, D), lambda i: (i, 0)),
        out_shape=jax.ShapeDtypeStruct((N_PAD, D), jnp.float32),
    )(pn, dvp, xt)[:N_NODES]


def _pad_idx(idx, pad_val):
    cols = TOTCH * CH - EPT
    return jnp.concatenate(
        [idx.reshape(NS, EPT),
         jnp.full((NS, cols), pad_val, jnp.int32)],
        axis=1).reshape(NS, TOTCH, CH)


def kernel(feat, node_idx, hedge_idx, W_in, b_in, W1, b1, W2, b2):
    f32 = jnp.float32
    feat_p = jnp.zeros((N_PAD, D), f32).at[:N_NODES, :].set(feat)
    node_p = _pad_idx(node_idx, PAD_NODE)
    hedge_p = _pad_idx(hedge_idx, PAD_HEDGE)
    zeros1 = jnp.zeros((N_PAD // L, L), f32)

    dvp, dep = _hist_call(node_p, hedge_p, zeros1)
    dvp = dvp.reshape(NW, N_PAD)
    dep = dep.reshape(NW, E_PAD)
    xt, ystack, demat = _tc1_call(feat_p, W_in, b_in.reshape(1, D), W2,
                                  b2.reshape(1, D), dvp, dep)
    pn = _main_call(ystack, demat, node_p, hedge_p)
    return _tc3_call(pn, dvp, xt)
